# baseline (speedup 1.0000x reference)
import functools

import jax
import jax.numpy as jnp
from jax import lax
from jax.experimental import pallas as pl
from jax.experimental.pallas import tpu as pltpu
from jax.experimental.pallas import tpu_sc as plsc

N = 10000
E = 640000
NIN = 128
H = 4
C = 32
HC = H * C

NC = 2
NS = 16
NW = NC * NS
EPW = E // NW
K1 = 400
NB1 = EPW // K1
SB = 80
NSB = EPW // SB
NPAD = 10240
ROWS_PER_TILE = NPAD // NS
RCHUNKS = ROWS_PER_TILE // SB
NDEEP = 4


def _tc_project_body(x_ref, w_ref, asw_ref, adw_ref, xw_ref, as_ref, ad_ref):
    xw = lax.dot_general(x_ref[...], w_ref[...], (((1,), (1,)), ((), ())),
                         preferred_element_type=jnp.float32)
    xw_ref[...] = xw
    for h in range(H):
        sl = xw[:, h * C:(h + 1) * C]
        as_ref[:, h:h + 1] = jnp.sum(sl * asw_ref[h:h + 1, :], axis=1,
                                     keepdims=True)
        ad_ref[:, h:h + 1] = jnp.sum(sl * adw_ref[h:h + 1, :], axis=1,
                                     keepdims=True)


def _tc_project(x, W, att_src, att_dst):
    blk = 2000
    grid = N // blk
    return pl.pallas_call(
        _tc_project_body,
        grid=(grid,),
        in_specs=[
            pl.BlockSpec((blk, NIN), lambda i: (i, 0)),
            pl.BlockSpec((HC, NIN), lambda i: (0, 0)),
            pl.BlockSpec((H, C), lambda i: (0, 0)),
            pl.BlockSpec((H, C), lambda i: (0, 0)),
        ],
        out_specs=[
            pl.BlockSpec((blk, HC), lambda i: (i, 0)),
            pl.BlockSpec((blk, H), lambda i: (i, 0)),
            pl.BlockSpec((blk, H), lambda i: (i, 0)),
        ],
        out_shape=[
            jax.ShapeDtypeStruct((N, HC), jnp.float32),
            jax.ShapeDtypeStruct((N, H), jnp.float32),
            jax.ShapeDtypeStruct((N, H), jnp.float32),
        ],
    )(x, W, att_src, att_dst)


def _tc_transpose_body(as_ref, ad_ref, i4_ref, ast_ref, adt_ref):
    i4 = i4_ref[...]
    dn = (((1,), (1,)), ((), ()))
    ast_ref[...] = lax.dot_general(i4, as_ref[...], dn,
                                   preferred_element_type=jnp.float32)
    adt_ref[...] = lax.dot_general(i4, ad_ref[...], dn,
                                   preferred_element_type=jnp.float32)


def _tc_transpose(a_src, a_dst, i4):
    return pl.pallas_call(
        _tc_transpose_body,
        out_shape=[
            jax.ShapeDtypeStruct((H, NPAD), jnp.float32),
            jax.ShapeDtypeStruct((H, NPAD), jnp.float32),
        ],
    )(a_src, a_dst, i4)


def _sc_phase1_body(asrc_hbm, adst_hbm, src_hbm, dst_hbm, e_hbm, denp_hbm,
                    asrc_v, adst_v, den_v, si0, si1, di0, di1, ec0, ec1,
                    sin0, sin1, so0, so1):
    cid = lax.axis_index("c")
    sid = lax.axis_index("s")
    wid = sid * NC + cid
    sib = (si0, si1)
    dib = (di0, di1)
    ecb = (ec0, ec1)
    sem_i = (sin0, sin1)
    sem_o = (so0, so1)

    for h in range(H):
        pltpu.sync_copy(asrc_hbm.at[h], asrc_v.at[pl.ds(h * NPAD, NPAD)])
        pltpu.sync_copy(adst_hbm.at[h], adst_v.at[pl.ds(h * NPAD, NPAD)])

    zeros16 = jnp.zeros((16,), jnp.float32)

    @pl.loop(0, (NPAD * H) // 16)
    def _zero(i):
        den_v[pl.ds(i * 16, 16)] = zeros16

    iota16 = lax.iota(jnp.int32, 16)

    def in_descs(j, b):
        base = wid * EPW + j * K1
        yield (src_hbm.at[pl.ds(base, K1)], sib[b])
        yield (dst_hbm.at[pl.ds(base, K1)], dib[b])

    def out_desc(j, b):
        base = wid * EPW + j * K1
        return (ecb[b], e_hbm.at[pl.ds(base * H, K1 * H)])

    def issue_in(j, b):
        for s_, d_ in in_descs(j, b):
            pltpu.async_copy(s_, d_, sem_i[b])

    issue_in(0, 0)

    @pl.loop(0, NB1, step=2)
    def _batch(i):
        for b in range(2):
            j = i + b

            @pl.when(j + 1 < NB1)
            def _():
                issue_in(j + 1, 1 - b)

            @pl.when(j >= 2)
            def _():
                s_, d_ = out_desc(j - 2, b)
                pltpu.make_async_copy(s_, d_, sem_o[b]).wait()

            for s_, d_ in in_descs(j, b):
                pltpu.make_async_copy(s_, d_, sem_i[b]).wait()

            sidx = sib[b]
            didx = dib[b]
            e_c = ecb[b]

            @pl.loop(0, K1 // 16, unroll=2)
            def _grp(jj):
                sv = sidx[pl.ds(jj * 16, 16)]
                dv = didx[pl.ds(jj * 16, 16)]
                kvec = jj * 16 + iota16
                for h in range(H):
                    a_s = plsc.load_gather(asrc_v, [sv + h * NPAD])
                    a_d = plsc.load_gather(adst_v, [dv + h * NPAD])
                    al = a_s + a_d
                    al = jnp.where(al >= 0.0, al, al * jnp.float32(0.2))
                    e = jnp.exp(al)
                    plsc.addupdate_scatter(den_v, [dv * H + h], e)
                    plsc.store_scatter(e_c, [kvec * H + h], e)

            s_, d_ = out_desc(j, b)
            pltpu.async_copy(s_, d_, sem_o[b])

    for j in (NB1 - 2, NB1 - 1):
        s_, d_ = out_desc(j, j % 2)
        pltpu.make_async_copy(s_, d_, sem_o[j % 2]).wait()

    pltpu.sync_copy(den_v, denp_hbm.at[wid])


def _sc_phase1(asrc_flat, adst_flat, src, dst):
    mesh = plsc.VectorSubcoreMesh(core_axis_name="c", subcore_axis_name="s")
    f = functools.partial(
        pl.kernel,
        out_type=(
            jax.ShapeDtypeStruct((E * H,), jnp.float32),
            jax.ShapeDtypeStruct((NW, NPAD * H), jnp.float32),
        ),
        mesh=mesh,
        scratch_types=[
            pltpu.VMEM((NPAD * H,), jnp.float32),
            pltpu.VMEM((NPAD * H,), jnp.float32),
            pltpu.VMEM((NPAD * H,), jnp.float32),
            pltpu.VMEM((K1,), jnp.int32),
            pltpu.VMEM((K1,), jnp.int32),
            pltpu.VMEM((K1,), jnp.int32),
            pltpu.VMEM((K1,), jnp.int32),
            pltpu.VMEM((K1 * H,), jnp.float32),
            pltpu.VMEM((K1 * H,), jnp.float32),
        ] + [pltpu.SemaphoreType.DMA] * 4,
        compiler_params=pltpu.CompilerParams(needs_layout_passes=False),
    )(_sc_phase1_body)
    return f(asrc_flat, adst_flat, src, dst)


def _sc_phase2_body(xwp_hbm, sd_hbm, ef_hbm, accp_hbm,
                    gbf0, gbf1, s0, s1, e0, e1, e2, e3,
                    sd0, sd1, sd2, sd3, acc,
                    *sems):
    cid = lax.axis_index("c")
    sid = lax.axis_index("s")
    wid = sid * NC + cid
    gbfb = (gbf0, gbf1)
    sb = (s0, s1)
    eb = (e0, e1, e2, e3)
    sdb = (sd0, sd1, sd2, sd3)
    sem_i = sems[0:NDEEP]
    sem_g = sems[NDEEP:NDEEP + 2]
    sem_s = sems[NDEEP + 2:NDEEP + 4]

    zeros16 = jnp.zeros((16,), jnp.float32)

    @pl.loop(0, SB)
    def _zg(r):
        for c8 in range(HC // 16):
            s0[r, pl.ds(c8 * 16, 16)] = zeros16

    row0 = sid * ROWS_PER_TILE
    for t in range(RCHUNKS):
        pltpu.sync_copy(s0.at[pl.ds(0, SB)],
                        acc.at[pl.ds(row0 + t * SB, SB)])
    plsc.subcore_barrier()

    def in_descs(j, b4):
        yield (sd_hbm.at[wid, j], sdb[b4])
        yield (ef_hbm.at[pl.ds((wid * EPW + j * SB) * H, SB * H)],
               eb[b4].at[pl.ds(0, SB * H)])

    def g_desc(j, b4, b2):
        del j
        return (xwp_hbm.at[sdb[b4].at[0]], gbfb[b2])

    def s_desc(j, b4, b2):
        del j
        return (sb[b2], acc.at[sdb[b4].at[1]])

    def issue_in(j, b4):
        for s_, d_ in in_descs(j, b4):
            pltpu.async_copy(s_, d_, sem_i[b4])

    def wait_in(j, b4):
        for s_, d_ in in_descs(j, b4):
            pltpu.make_async_copy(s_, d_, sem_i[b4]).wait()

    issue_in(0, 0)
    issue_in(1, 1)
    wait_in(0, 0)
    s_, d_ = g_desc(0, 0, 0)
    pltpu.async_copy(s_, d_, sem_g[0])

    @pl.loop(0, NSB + 2, step=NDEEP)
    def _sb(i):
        for b in range(NDEEP):
            j = i + b
            b2 = b % 2
            bn2_2 = b2
            bn1_4 = (b + 1) % NDEEP
            bn2_4 = (b + 2) % NDEEP
            bn1_2 = (b + 1) % 2

            @pl.when(j >= 2)
            def _():
                s_, d_ = s_desc(j - 2, bn2_4, bn2_2)
                pltpu.make_async_copy(s_, d_, sem_s[bn2_2]).wait()

            @pl.when(j + 2 < NSB)
            def _():
                issue_in(j + 2, bn2_4)

            @pl.when(j + 1 < NSB)
            def _():
                wait_in(j + 1, bn1_4)
                s_, d_ = g_desc(j + 1, bn1_4, bn1_2)
                pltpu.async_copy(s_, d_, sem_g[bn1_2])

            @pl.when(j < NSB)
            def _():
                s_, d_ = g_desc(j, b, b2)
                pltpu.make_async_copy(s_, d_, sem_g[b2]).wait()

                gbf = gbfb[b2]
                s = sb[b2]
                ev_ref = eb[b]

                @pl.loop(0, SB, unroll=2)
                def _edge(k):
                    ev = ev_ref[pl.ds(k * H, 16)]
                    for h in range(H):
                        bvec = jnp.full((16,), ev[h], jnp.float32)
                        for half in range(C // 16):
                            col = h * C + half * 16
                            s[k, pl.ds(col, 16)] = gbf[k, pl.ds(col, 16)] * bvec

                s_, d_ = s_desc(j, b, b2)
                pltpu.async_copy(s_, d_, sem_s[b2], add=True)

    plsc.subcore_barrier()

    for t in range(RCHUNKS):
        pltpu.sync_copy(acc.at[pl.ds(row0 + t * SB, SB)], s0.at[pl.ds(0, SB)])
        pltpu.sync_copy(s0.at[pl.ds(0, SB)],
                        accp_hbm.at[cid, pl.ds(row0 + t * SB, SB)])


def _sc_phase2(xwp, sd, e_all):
    mesh = plsc.VectorSubcoreMesh(core_axis_name="c", subcore_axis_name="s")
    f = functools.partial(
        pl.kernel,
        out_type=jax.ShapeDtypeStruct((NC, NPAD, HC), jnp.float32),
        mesh=mesh,
        scratch_types=(
            [pltpu.VMEM((SB, HC), jnp.float32)] * 2
            + [pltpu.VMEM((SB, HC), jnp.float32)] * 2
            + [pltpu.VMEM((SB * H + 16,), jnp.float32)] * NDEEP
            + [pltpu.VMEM((2, SB), jnp.int32)] * NDEEP
            + [pltpu.VMEM_SHARED((NPAD, HC), jnp.float32)]
            + [pltpu.SemaphoreType.DMA] * (NDEEP + 4)
        ),
        compiler_params=pltpu.CompilerParams(needs_layout_passes=False),
    )(_sc_phase2_body)
    return f(xwp, sd, e_all)


def _tc_densum_body(dp_ref, out_ref):
    out_ref[...] = jnp.sum(dp_ref[...], axis=0)


def _tc_densum(denp3):
    rows = NPAD * H // 128
    blkr = 40
    return pl.pallas_call(
        _tc_densum_body,
        grid=(rows // blkr,),
        in_specs=[pl.BlockSpec((NW, blkr, 128), lambda i: (0, i, 0))],
        out_specs=pl.BlockSpec((blkr, 128), lambda i: (i, 0)),
        out_shape=jax.ShapeDtypeStruct((rows, 128), jnp.float32),
    )(denp3)


def _tc_fin_body(acc_ref, den_ref, as_ref, ad_ref, xw_ref, b_ref, out_ref):
    acc = acc_ref[0] + acc_ref[1]
    den = den_ref[...]
    al = as_ref[...] + ad_ref[...]
    al = jnp.where(al >= 0.0, al, al * jnp.float32(0.2))
    es = jnp.exp(al)
    xw = xw_ref[...]
    for h in range(H):
        sl = slice(h * C, (h + 1) * C)
        num = acc[:, sl] + es[:, h:h + 1] * xw[:, sl]
        d = den[:, h:h + 1] + es[:, h:h + 1] + jnp.float32(1e-16)
        out_ref[:, sl] = num / d + b_ref[0, sl]


def _tc_fin(accp, den_tot, a_src, a_dst, xw, bias):
    blk = 2000
    grid = N // blk
    return pl.pallas_call(
        _tc_fin_body,
        grid=(grid,),
        in_specs=[
            pl.BlockSpec((NC, blk, HC), lambda i: (0, i, 0)),
            pl.BlockSpec((blk, H), lambda i: (i, 0)),
            pl.BlockSpec((blk, H), lambda i: (i, 0)),
            pl.BlockSpec((blk, H), lambda i: (i, 0)),
            pl.BlockSpec((blk, HC), lambda i: (i, 0)),
            pl.BlockSpec((1, HC), lambda i: (0, 0)),
        ],
        out_specs=pl.BlockSpec((blk, HC), lambda i: (i, 0)),
        out_shape=jax.ShapeDtypeStruct((N, HC), jnp.float32),
    )(accp, den_tot, a_src, a_dst, xw, bias)


def kernel(x, edge_index, edge_attr, W, att_src, att_dst, bias):
    del edge_attr
    src = edge_index[0].astype(jnp.int32)
    dst = edge_index[1].astype(jnp.int32)
    xw, a_src, a_dst = _tc_project(x, W, att_src, att_dst)
    pad = ((0, NPAD - N), (0, 0))
    a_srcT, a_dstT = _tc_transpose(jnp.pad(a_src, pad), jnp.pad(a_dst, pad),
                                   jnp.eye(H, dtype=jnp.float32))
    e_all, denp = _sc_phase1(a_srcT, a_dstT, src, dst)
    sd = jnp.stack([src.reshape(NW, NSB, SB), dst.reshape(NW, NSB, SB)],
                   axis=2)
    accp = _sc_phase2(xw, sd, e_all)
    den_tot = _tc_densum(denp.reshape(NW, NPAD * H // 128, 128))
    out = _tc_fin(accp, den_tot.reshape(NPAD, H), a_src, a_dst, xw,
                  bias.reshape(1, HC))
    return out


# speedup vs baseline: 2.0023x; 2.0023x over previous
import functools

import jax
import jax.numpy as jnp
from jax import lax
from jax.experimental import pallas as pl
from jax.experimental.pallas import tpu as pltpu
from jax.experimental.pallas import tpu_sc as plsc

N = 10000
E = 640000
NIN = 128
H = 4
C = 32
HC = H * C

NC = 2
NS = 16
NW = NC * NS
EPW = E // NW
K1 = 400
NB1 = EPW // K1
SB = 80
NSB = EPW // SB
NPAD = 10240
ROWS_PER_TILE = NPAD // NS
RCHUNKS = ROWS_PER_TILE // SB
NDEEP = 4


def _tc_project_body(x_ref, w_ref, asw_ref, adw_ref, xw_ref, as_ref, ad_ref):
    xw = lax.dot_general(x_ref[...], w_ref[...], (((1,), (1,)), ((), ())),
                         preferred_element_type=jnp.float32)
    xw_ref[...] = xw
    for h in range(H):
        sl = xw[:, h * C:(h + 1) * C]
        as_ref[:, h:h + 1] = jnp.sum(sl * asw_ref[h:h + 1, :], axis=1,
                                     keepdims=True)
        ad_ref[:, h:h + 1] = jnp.sum(sl * adw_ref[h:h + 1, :], axis=1,
                                     keepdims=True)


def _tc_project(x, W, att_src, att_dst):
    blk = 2000
    grid = N // blk
    return pl.pallas_call(
        _tc_project_body,
        grid=(grid,),
        in_specs=[
            pl.BlockSpec((blk, NIN), lambda i: (i, 0)),
            pl.BlockSpec((HC, NIN), lambda i: (0, 0)),
            pl.BlockSpec((H, C), lambda i: (0, 0)),
            pl.BlockSpec((H, C), lambda i: (0, 0)),
        ],
        out_specs=[
            pl.BlockSpec((blk, HC), lambda i: (i, 0)),
            pl.BlockSpec((blk, H), lambda i: (i, 0)),
            pl.BlockSpec((blk, H), lambda i: (i, 0)),
        ],
        out_shape=[
            jax.ShapeDtypeStruct((N, HC), jnp.float32),
            jax.ShapeDtypeStruct((N, H), jnp.float32),
            jax.ShapeDtypeStruct((N, H), jnp.float32),
        ],
    )(x, W, att_src, att_dst)


def _tc_transpose_body(as_ref, ad_ref, i4_ref, ast_ref, adt_ref):
    i4 = i4_ref[...]
    dn = (((1,), (1,)), ((), ()))
    ast_ref[...] = lax.dot_general(i4, as_ref[...], dn,
                                   preferred_element_type=jnp.float32)
    adt_ref[...] = lax.dot_general(i4, ad_ref[...], dn,
                                   preferred_element_type=jnp.float32)


def _tc_transpose(a_src, a_dst, i4):
    return pl.pallas_call(
        _tc_transpose_body,
        out_shape=[
            jax.ShapeDtypeStruct((H, NPAD), jnp.float32),
            jax.ShapeDtypeStruct((H, NPAD), jnp.float32),
        ],
    )(a_src, a_dst, i4)


def _sc_phase1_body(asrc_hbm, adst_hbm, src_hbm, dst_hbm, e_hbm, denp_hbm,
                    asrc_v, adst_v, den_v, si0, si1, di0, di1, ec0, ec1,
                    sin0, sin1, so0, so1):
    cid = lax.axis_index("c")
    sid = lax.axis_index("s")
    wid = sid * NC + cid
    sib = (si0, si1)
    dib = (di0, di1)
    ecb = (ec0, ec1)
    sem_i = (sin0, sin1)
    sem_o = (so0, so1)

    for h in range(H):
        pltpu.sync_copy(asrc_hbm.at[h], asrc_v.at[pl.ds(h * NPAD, NPAD)])
        pltpu.sync_copy(adst_hbm.at[h], adst_v.at[pl.ds(h * NPAD, NPAD)])

    zeros16 = jnp.zeros((16,), jnp.float32)

    @pl.loop(0, (NPAD * H) // 16)
    def _zero(i):
        den_v[pl.ds(i * 16, 16)] = zeros16

    iota16 = lax.iota(jnp.int32, 16)

    def in_descs(j, b):
        base = wid * EPW + j * K1
        yield (src_hbm.at[pl.ds(base, K1)], sib[b])
        yield (dst_hbm.at[pl.ds(base, K1)], dib[b])

    def out_desc(j, b):
        base = wid * EPW + j * K1
        return (ecb[b], e_hbm.at[pl.ds(base * H, K1 * H)])

    def issue_in(j, b):
        for s_, d_ in in_descs(j, b):
            pltpu.async_copy(s_, d_, sem_i[b])

    issue_in(0, 0)

    @pl.loop(0, NB1, step=2)
    def _batch(i):
        for b in range(2):
            j = i + b

            @pl.when(j + 1 < NB1)
            def _():
                issue_in(j + 1, 1 - b)

            @pl.when(j >= 2)
            def _():
                s_, d_ = out_desc(j - 2, b)
                pltpu.make_async_copy(s_, d_, sem_o[b]).wait()

            for s_, d_ in in_descs(j, b):
                pltpu.make_async_copy(s_, d_, sem_i[b]).wait()

            sidx = sib[b]
            didx = dib[b]
            e_c = ecb[b]

            @pl.loop(0, K1 // 16, unroll=2)
            def _grp(jj):
                sv = sidx[pl.ds(jj * 16, 16)]
                dv = didx[pl.ds(jj * 16, 16)]
                kvec = jj * 16 + iota16
                for h in range(H):
                    a_s = plsc.load_gather(asrc_v, [sv + h * NPAD])
                    a_d = plsc.load_gather(adst_v, [dv + h * NPAD])
                    al = a_s + a_d
                    al = jnp.where(al >= 0.0, al, al * jnp.float32(0.2))
                    e = jnp.exp(al)
                    plsc.addupdate_scatter(den_v, [dv * H + h], e)
                    plsc.store_scatter(e_c, [kvec * H + h], e)

            s_, d_ = out_desc(j, b)
            pltpu.async_copy(s_, d_, sem_o[b])

    for j in (NB1 - 2, NB1 - 1):
        s_, d_ = out_desc(j, j % 2)
        pltpu.make_async_copy(s_, d_, sem_o[j % 2]).wait()

    pltpu.sync_copy(den_v, denp_hbm.at[wid])


def _sc_phase1(asrc_flat, adst_flat, src, dst):
    mesh = plsc.VectorSubcoreMesh(core_axis_name="c", subcore_axis_name="s")
    f = functools.partial(
        pl.kernel,
        out_type=(
            jax.ShapeDtypeStruct((E * H,), jnp.float32),
            jax.ShapeDtypeStruct((NW, NPAD * H), jnp.float32),
        ),
        mesh=mesh,
        scratch_types=[
            pltpu.VMEM((NPAD * H,), jnp.float32),
            pltpu.VMEM((NPAD * H,), jnp.float32),
            pltpu.VMEM((NPAD * H,), jnp.float32),
            pltpu.VMEM((K1,), jnp.int32),
            pltpu.VMEM((K1,), jnp.int32),
            pltpu.VMEM((K1,), jnp.int32),
            pltpu.VMEM((K1,), jnp.int32),
            pltpu.VMEM((K1 * H,), jnp.float32),
            pltpu.VMEM((K1 * H,), jnp.float32),
        ] + [pltpu.SemaphoreType.DMA] * 4,
        compiler_params=pltpu.CompilerParams(needs_layout_passes=False),
    )(_sc_phase1_body)
    return f(asrc_flat, adst_flat, src, dst)


def _sc_phase2_body(xw_hbm, sd_hbm, ef_hbm, accp_hbm,
                    g0, g1, g2, g3, e0, e1, e2, e3,
                    sd0, sd1, sd2, sd3, acc,
                    *sems):
    cid = lax.axis_index("c")
    sid = lax.axis_index("s")
    wid = sid * NC + cid
    gb = (g0, g1, g2, g3)
    eb = (e0, e1, e2, e3)
    sdb = (sd0, sd1, sd2, sd3)
    sem_i = sems[0:NDEEP]
    sem_g = sems[NDEEP:2 * NDEEP]
    sem_s = sems[2 * NDEEP:3 * NDEEP]

    zeros16 = jnp.zeros((16,), jnp.float32)

    @pl.loop(0, SB)
    def _zg(r):
        for c8 in range(HC // 16):
            g0[r, pl.ds(c8 * 16, 16)] = zeros16

    row0 = sid * ROWS_PER_TILE
    for t in range(RCHUNKS):
        pltpu.sync_copy(g0.at[pl.ds(0, SB)],
                        acc.at[pl.ds(row0 + t * SB, SB)])
    plsc.subcore_barrier()

    def in_descs(j, b):
        yield (sd_hbm.at[wid, j], sdb[b])
        yield (ef_hbm.at[pl.ds((wid * EPW + j * SB) * H, SB * H)],
               eb[b].at[pl.ds(0, SB * H)])

    def g_desc(j, b):
        del j
        return (xw_hbm.at[sdb[b].at[0]], gb[b])

    def s_desc(j, b):
        del j
        return (gb[b], acc.at[sdb[b].at[1]])

    def issue_in(j, b):
        for s_, d_ in in_descs(j, b):
            pltpu.async_copy(s_, d_, sem_i[b])

    def wait_in(j, b):
        for s_, d_ in in_descs(j, b):
            pltpu.make_async_copy(s_, d_, sem_i[b]).wait()

    issue_in(0, 0)
    issue_in(1, 1)
    wait_in(0, 0)
    s_, d_ = g_desc(0, 0)
    pltpu.async_copy(s_, d_, sem_g[0])

    @pl.loop(0, NSB + 2, step=NDEEP)
    def _sb(i):
        for b in range(NDEEP):
            j = i + b
            bn1 = (b + 1) % NDEEP
            bn2 = (b + 2) % NDEEP

            @pl.when(j >= 2)
            def _():
                s_, d_ = s_desc(j - 2, bn2)
                pltpu.make_async_copy(s_, d_, sem_s[bn2]).wait()

            @pl.when(j + 2 < NSB)
            def _():
                issue_in(j + 2, bn2)

            @pl.when(j + 1 < NSB)
            def _():
                wait_in(j + 1, bn1)
                s_, d_ = g_desc(j + 1, bn1)
                pltpu.async_copy(s_, d_, sem_g[bn1])

            @pl.when(j < NSB)
            def _():
                s_, d_ = g_desc(j, b)
                pltpu.make_async_copy(s_, d_, sem_g[b]).wait()

                g = gb[b]
                ev_ref = eb[b]

                @pl.loop(0, SB, unroll=2)
                def _edge(k):
                    ev = ev_ref[pl.ds(k * H, 16)]
                    for h in range(H):
                        bvec = jnp.full((16,), ev[h], jnp.float32)
                        for half in range(C // 16):
                            col = h * C + half * 16
                            g[k, pl.ds(col, 16)] = g[k, pl.ds(col, 16)] * bvec

                s_, d_ = s_desc(j, b)
                pltpu.async_copy(s_, d_, sem_s[b], add=True)

    plsc.subcore_barrier()

    for t in range(RCHUNKS):
        pltpu.sync_copy(acc.at[pl.ds(row0 + t * SB, SB)], g0.at[pl.ds(0, SB)])
        pltpu.sync_copy(g0.at[pl.ds(0, SB)],
                        accp_hbm.at[cid, pl.ds(row0 + t * SB, SB)])


def _sc_phase2(xw, sd, e_all):
    mesh = plsc.VectorSubcoreMesh(core_axis_name="c", subcore_axis_name="s")
    f = functools.partial(
        pl.kernel,
        out_type=jax.ShapeDtypeStruct((NC, NPAD, HC), jnp.float32),
        mesh=mesh,
        scratch_types=(
            [pltpu.VMEM((SB, HC), jnp.float32)] * NDEEP
            + [pltpu.VMEM((SB * H + 16,), jnp.float32)] * NDEEP
            + [pltpu.VMEM((2, SB), jnp.int32)] * NDEEP
            + [pltpu.VMEM_SHARED((NPAD, HC), jnp.float32)]
            + [pltpu.SemaphoreType.DMA] * (3 * NDEEP)
        ),
        compiler_params=pltpu.CompilerParams(needs_layout_passes=False),
    )(_sc_phase2_body)
    return f(xw, sd, e_all)


def _tc_densum_body(dp_ref, out_ref):
    out_ref[...] = jnp.sum(dp_ref[...], axis=0)


def _tc_densum(denp3):
    rows = NPAD * H // 128
    blkr = 40
    return pl.pallas_call(
        _tc_densum_body,
        grid=(rows // blkr,),
        in_specs=[pl.BlockSpec((NW, blkr, 128), lambda i: (0, i, 0))],
        out_specs=pl.BlockSpec((blkr, 128), lambda i: (i, 0)),
        out_shape=jax.ShapeDtypeStruct((rows, 128), jnp.float32),
    )(denp3)


def _tc_fin_body(acc_ref, den_ref, as_ref, ad_ref, xw_ref, b_ref, out_ref):
    acc = acc_ref[0] + acc_ref[1]
    den = den_ref[...]
    al = as_ref[...] + ad_ref[...]
    al = jnp.where(al >= 0.0, al, al * jnp.float32(0.2))
    es = jnp.exp(al)
    xw = xw_ref[...]
    for h in range(H):
        sl = slice(h * C, (h + 1) * C)
        num = acc[:, sl] + es[:, h:h + 1] * xw[:, sl]
        d = den[:, h:h + 1] + es[:, h:h + 1] + jnp.float32(1e-16)
        out_ref[:, sl] = num / d + b_ref[0, sl]


def _tc_fin(accp, den_tot, a_src, a_dst, xw, bias):
    blk = 2000
    grid = N // blk
    return pl.pallas_call(
        _tc_fin_body,
        grid=(grid,),
        in_specs=[
            pl.BlockSpec((NC, blk, HC), lambda i: (0, i, 0)),
            pl.BlockSpec((blk, H), lambda i: (i, 0)),
            pl.BlockSpec((blk, H), lambda i: (i, 0)),
            pl.BlockSpec((blk, H), lambda i: (i, 0)),
            pl.BlockSpec((blk, HC), lambda i: (i, 0)),
            pl.BlockSpec((1, HC), lambda i: (0, 0)),
        ],
        out_specs=pl.BlockSpec((blk, HC), lambda i: (i, 0)),
        out_shape=jax.ShapeDtypeStruct((N, HC), jnp.float32),
    )(accp, den_tot, a_src, a_dst, xw, bias)


def kernel(x, edge_index, edge_attr, W, att_src, att_dst, bias):
    del edge_attr
    src = edge_index[0].astype(jnp.int32)
    dst = edge_index[1].astype(jnp.int32)
    xw, a_src, a_dst = _tc_project(x, W, att_src, att_dst)
    pad = ((0, NPAD - N), (0, 0))
    a_srcT, a_dstT = _tc_transpose(jnp.pad(a_src, pad), jnp.pad(a_dst, pad),
                                   jnp.eye(H, dtype=jnp.float32))
    e_all, denp = _sc_phase1(a_srcT, a_dstT, src, dst)
    sd = jnp.stack([src.reshape(NW, NSB, SB), dst.reshape(NW, NSB, SB)],
                   axis=2)
    accp = _sc_phase2(xw, sd, e_all)
    den_tot = _tc_densum(denp.reshape(NW, NPAD * H // 128, 128))
    out = _tc_fin(accp, den_tot.reshape(NPAD, H), a_src, a_dst, xw,
                  bias.reshape(1, HC))
    return out


# speedup vs baseline: 2.0276x; 1.0126x over previous
import functools

import jax
import jax.numpy as jnp
from jax import lax
from jax.experimental import pallas as pl
from jax.experimental.pallas import tpu as pltpu
from jax.experimental.pallas import tpu_sc as plsc

N = 10000
E = 640000
NIN = 128
H = 4
C = 32
HC = H * C

NC = 2
NS = 16
NW = NC * NS
EPW = E // NW
K1 = 400
NB1 = EPW // K1
SB = 80
NSB = EPW // SB
NPAD = 10240
ROWS_PER_TILE = NPAD // NS
RCHUNKS = ROWS_PER_TILE // SB
NDEEP = 4


def _tc_project_body(x_ref, w_ref, asw_ref, adw_ref, xw_ref, as_ref, ad_ref):
    xw = lax.dot_general(x_ref[...], w_ref[...], (((1,), (1,)), ((), ())),
                         preferred_element_type=jnp.float32)
    xw_ref[...] = xw
    for h in range(H):
        sl = xw[:, h * C:(h + 1) * C]
        as_ref[:, h:h + 1] = jnp.sum(sl * asw_ref[h:h + 1, :], axis=1,
                                     keepdims=True)
        ad_ref[:, h:h + 1] = jnp.sum(sl * adw_ref[h:h + 1, :], axis=1,
                                     keepdims=True)


def _tc_project(x, W, att_src, att_dst):
    blk = 2000
    grid = N // blk
    return pl.pallas_call(
        _tc_project_body,
        grid=(grid,),
        in_specs=[
            pl.BlockSpec((blk, NIN), lambda i: (i, 0)),
            pl.BlockSpec((HC, NIN), lambda i: (0, 0)),
            pl.BlockSpec((H, C), lambda i: (0, 0)),
            pl.BlockSpec((H, C), lambda i: (0, 0)),
        ],
        out_specs=[
            pl.BlockSpec((blk, HC), lambda i: (i, 0)),
            pl.BlockSpec((blk, H), lambda i: (i, 0)),
            pl.BlockSpec((blk, H), lambda i: (i, 0)),
        ],
        out_shape=[
            jax.ShapeDtypeStruct((N, HC), jnp.float32),
            jax.ShapeDtypeStruct((N, H), jnp.float32),
            jax.ShapeDtypeStruct((N, H), jnp.float32),
        ],
    )(x, W, att_src, att_dst)


def _tc_transpose_body(as_ref, ad_ref, i4_ref, ast_ref, adt_ref):
    i4 = i4_ref[...]
    dn = (((1,), (1,)), ((), ()))
    ast_ref[...] = lax.dot_general(i4, as_ref[...], dn,
                                   preferred_element_type=jnp.float32)
    adt_ref[...] = lax.dot_general(i4, ad_ref[...], dn,
                                   preferred_element_type=jnp.float32)


def _tc_transpose(a_src, a_dst, i4):
    return pl.pallas_call(
        _tc_transpose_body,
        out_shape=[
            jax.ShapeDtypeStruct((H, NPAD), jnp.float32),
            jax.ShapeDtypeStruct((H, NPAD), jnp.float32),
        ],
    )(a_src, a_dst, i4)


def _sc_phase1_body(asrc_hbm, adst_hbm, src_hbm, dst_hbm, e_hbm, denp_hbm,
                    asrc_v, adst_v, den_v, si0, si1, di0, di1, ec0, ec1,
                    sin0, sin1, so0, so1):
    cid = lax.axis_index("c")
    sid = lax.axis_index("s")
    wid = sid * NC + cid
    sib = (si0, si1)
    dib = (di0, di1)
    ecb = (ec0, ec1)
    sem_i = (sin0, sin1)
    sem_o = (so0, so1)

    for h in range(H):
        pltpu.async_copy(asrc_hbm.at[h], asrc_v.at[pl.ds(h * NPAD, NPAD)],
                         sin0)
        pltpu.async_copy(adst_hbm.at[h], adst_v.at[pl.ds(h * NPAD, NPAD)],
                         sin0)
    for h in range(H):
        pltpu.make_async_copy(asrc_hbm.at[h],
                              asrc_v.at[pl.ds(h * NPAD, NPAD)], sin0).wait()
        pltpu.make_async_copy(adst_hbm.at[h],
                              adst_v.at[pl.ds(h * NPAD, NPAD)], sin0).wait()

    zeros16 = jnp.zeros((16,), jnp.float32)

    @pl.loop(0, (NPAD * H) // 16)
    def _zero(i):
        den_v[pl.ds(i * 16, 16)] = zeros16

    iota16 = lax.iota(jnp.int32, 16)

    def in_descs(j, b):
        base = wid * EPW + j * K1
        yield (src_hbm.at[pl.ds(base, K1)], sib[b])
        yield (dst_hbm.at[pl.ds(base, K1)], dib[b])

    def out_desc(j, b):
        base = wid * EPW + j * K1
        return (ecb[b], e_hbm.at[pl.ds(base * H, K1 * H)])

    def issue_in(j, b):
        for s_, d_ in in_descs(j, b):
            pltpu.async_copy(s_, d_, sem_i[b])

    issue_in(0, 0)

    @pl.loop(0, NB1, step=2)
    def _batch(i):
        for b in range(2):
            j = i + b

            @pl.when(j + 1 < NB1)
            def _():
                issue_in(j + 1, 1 - b)

            @pl.when(j >= 2)
            def _():
                s_, d_ = out_desc(j - 2, b)
                pltpu.make_async_copy(s_, d_, sem_o[b]).wait()

            for s_, d_ in in_descs(j, b):
                pltpu.make_async_copy(s_, d_, sem_i[b]).wait()

            sidx = sib[b]
            didx = dib[b]
            e_c = ecb[b]

            @pl.loop(0, K1 // 16, unroll=2)
            def _grp(jj):
                sv = sidx[pl.ds(jj * 16, 16)]
                dv = didx[pl.ds(jj * 16, 16)]
                kvec = jj * 16 + iota16
                for h in range(H):
                    a_s = plsc.load_gather(asrc_v, [sv + h * NPAD])
                    a_d = plsc.load_gather(adst_v, [dv + h * NPAD])
                    al = a_s + a_d
                    al = jnp.where(al >= 0.0, al, al * jnp.float32(0.2))
                    e = jnp.exp(al)
                    plsc.addupdate_scatter(den_v, [dv * H + h], e)
                    plsc.store_scatter(e_c, [kvec * H + h], e)

            s_, d_ = out_desc(j, b)
            pltpu.async_copy(s_, d_, sem_o[b])

    for j in (NB1 - 2, NB1 - 1):
        s_, d_ = out_desc(j, j % 2)
        pltpu.make_async_copy(s_, d_, sem_o[j % 2]).wait()

    pltpu.sync_copy(den_v, denp_hbm.at[wid])


def _sc_phase1(asrc_flat, adst_flat, src, dst):
    mesh = plsc.VectorSubcoreMesh(core_axis_name="c", subcore_axis_name="s")
    f = functools.partial(
        pl.kernel,
        out_type=(
            jax.ShapeDtypeStruct((E * H,), jnp.float32),
            jax.ShapeDtypeStruct((NW, NPAD * H), jnp.float32),
        ),
        mesh=mesh,
        scratch_types=[
            pltpu.VMEM((NPAD * H,), jnp.float32),
            pltpu.VMEM((NPAD * H,), jnp.float32),
            pltpu.VMEM((NPAD * H,), jnp.float32),
            pltpu.VMEM((K1,), jnp.int32),
            pltpu.VMEM((K1,), jnp.int32),
            pltpu.VMEM((K1,), jnp.int32),
            pltpu.VMEM((K1,), jnp.int32),
            pltpu.VMEM((K1 * H,), jnp.float32),
            pltpu.VMEM((K1 * H,), jnp.float32),
        ] + [pltpu.SemaphoreType.DMA] * 4,
        compiler_params=pltpu.CompilerParams(needs_layout_passes=False),
    )(_sc_phase1_body)
    return f(asrc_flat, adst_flat, src, dst)


def _sc_phase2_body(xw_hbm, sd_hbm, ef_hbm, accp_hbm,
                    g0, g1, g2, g3, e0, e1, e2, e3,
                    sd0, sd1, sd2, sd3, acc,
                    *sems):
    cid = lax.axis_index("c")
    sid = lax.axis_index("s")
    wid = sid * NC + cid
    gb = (g0, g1, g2, g3)
    eb = (e0, e1, e2, e3)
    sdb = (sd0, sd1, sd2, sd3)
    sem_i = sems[0:NDEEP]
    sem_g = sems[NDEEP:2 * NDEEP]
    sem_s = sems[2 * NDEEP:3 * NDEEP]

    zeros16 = jnp.zeros((16,), jnp.float32)

    @pl.loop(0, SB)
    def _zg(r):
        for c8 in range(HC // 16):
            g0[r, pl.ds(c8 * 16, 16)] = zeros16

    row0 = sid * ROWS_PER_TILE
    for t in range(RCHUNKS):
        pltpu.async_copy(g0.at[pl.ds(0, SB)],
                         acc.at[pl.ds(row0 + t * SB, SB)], sems[0])
    for t in range(RCHUNKS):
        pltpu.make_async_copy(g0.at[pl.ds(0, SB)],
                              acc.at[pl.ds(row0 + t * SB, SB)],
                              sems[0]).wait()
    plsc.subcore_barrier()

    def in_descs(j, b):
        yield (sd_hbm.at[wid, j], sdb[b])
        yield (ef_hbm.at[pl.ds((wid * EPW + j * SB) * H, SB * H)],
               eb[b].at[pl.ds(0, SB * H)])

    def g_desc(j, b):
        del j
        return (xw_hbm.at[sdb[b].at[0]], gb[b])

    def s_desc(j, b):
        del j
        return (gb[b], acc.at[sdb[b].at[1]])

    def issue_in(j, b):
        for s_, d_ in in_descs(j, b):
            pltpu.async_copy(s_, d_, sem_i[b])

    def wait_in(j, b):
        for s_, d_ in in_descs(j, b):
            pltpu.make_async_copy(s_, d_, sem_i[b]).wait()

    issue_in(0, 0)
    issue_in(1, 1)
    wait_in(0, 0)
    s_, d_ = g_desc(0, 0)
    pltpu.async_copy(s_, d_, sem_g[0])

    @pl.loop(0, NSB + 2, step=NDEEP)
    def _sb(i):
        for b in range(NDEEP):
            j = i + b
            bn1 = (b + 1) % NDEEP
            bn2 = (b + 2) % NDEEP

            @pl.when(j >= 2)
            def _():
                s_, d_ = s_desc(j - 2, bn2)
                pltpu.make_async_copy(s_, d_, sem_s[bn2]).wait()

            @pl.when(j + 2 < NSB)
            def _():
                issue_in(j + 2, bn2)

            @pl.when(j + 1 < NSB)
            def _():
                wait_in(j + 1, bn1)
                s_, d_ = g_desc(j + 1, bn1)
                pltpu.async_copy(s_, d_, sem_g[bn1])

            @pl.when(j < NSB)
            def _():
                s_, d_ = g_desc(j, b)
                pltpu.make_async_copy(s_, d_, sem_g[b]).wait()

                g = gb[b]
                ev_ref = eb[b]

                @pl.loop(0, SB, unroll=4)
                def _edge(k):
                    ev = ev_ref[pl.ds(k * H, 16)]
                    for h in range(H):
                        bvec = jnp.full((16,), ev[h], jnp.float32)
                        for half in range(C // 16):
                            col = h * C + half * 16
                            g[k, pl.ds(col, 16)] = g[k, pl.ds(col, 16)] * bvec

                s_, d_ = s_desc(j, b)
                pltpu.async_copy(s_, d_, sem_s[b], add=True)

    plsc.subcore_barrier()

    for t in range(RCHUNKS):
        gt = gb[t % 2]
        if t >= 2:
            pltpu.make_async_copy(
                gb[t % 2].at[pl.ds(0, SB)],
                accp_hbm.at[cid, pl.ds(row0 + (t - 2) * SB, SB)],
                sems[1]).wait()
        pltpu.sync_copy(acc.at[pl.ds(row0 + t * SB, SB)], gt.at[pl.ds(0, SB)])
        pltpu.async_copy(gt.at[pl.ds(0, SB)],
                         accp_hbm.at[cid, pl.ds(row0 + t * SB, SB)], sems[1])
    for t in (RCHUNKS - 2, RCHUNKS - 1):
        pltpu.make_async_copy(gb[t % 2].at[pl.ds(0, SB)],
                              accp_hbm.at[cid, pl.ds(row0 + t * SB, SB)],
                              sems[1]).wait()


def _sc_phase2(xw, sd, e_all):
    mesh = plsc.VectorSubcoreMesh(core_axis_name="c", subcore_axis_name="s")
    f = functools.partial(
        pl.kernel,
        out_type=jax.ShapeDtypeStruct((NC, NPAD, HC), jnp.float32),
        mesh=mesh,
        scratch_types=(
            [pltpu.VMEM((SB, HC), jnp.float32)] * NDEEP
            + [pltpu.VMEM((SB * H + 16,), jnp.float32)] * NDEEP
            + [pltpu.VMEM((2, SB), jnp.int32)] * NDEEP
            + [pltpu.VMEM_SHARED((NPAD, HC), jnp.float32)]
            + [pltpu.SemaphoreType.DMA] * (3 * NDEEP)
        ),
        compiler_params=pltpu.CompilerParams(needs_layout_passes=False),
    )(_sc_phase2_body)
    return f(xw, sd, e_all)


def _tc_densum_body(dp_ref, out_ref):
    out_ref[...] = jnp.sum(dp_ref[...], axis=0)


def _tc_densum(denp3):
    rows = NPAD * H // 128
    blkr = 40
    return pl.pallas_call(
        _tc_densum_body,
        grid=(rows // blkr,),
        in_specs=[pl.BlockSpec((NW, blkr, 128), lambda i: (0, i, 0))],
        out_specs=pl.BlockSpec((blkr, 128), lambda i: (i, 0)),
        out_shape=jax.ShapeDtypeStruct((rows, 128), jnp.float32),
    )(denp3)


def _tc_fin_body(acc_ref, den_ref, as_ref, ad_ref, xw_ref, b_ref, out_ref):
    acc = acc_ref[0] + acc_ref[1]
    den = den_ref[...]
    al = as_ref[...] + ad_ref[...]
    al = jnp.where(al >= 0.0, al, al * jnp.float32(0.2))
    es = jnp.exp(al)
    xw = xw_ref[...]
    for h in range(H):
        sl = slice(h * C, (h + 1) * C)
        num = acc[:, sl] + es[:, h:h + 1] * xw[:, sl]
        d = den[:, h:h + 1] + es[:, h:h + 1] + jnp.float32(1e-16)
        out_ref[:, sl] = num / d + b_ref[0, sl]


def _tc_fin(accp, den_tot, a_src, a_dst, xw, bias):
    blk = 2000
    grid = N // blk
    return pl.pallas_call(
        _tc_fin_body,
        grid=(grid,),
        in_specs=[
            pl.BlockSpec((NC, blk, HC), lambda i: (0, i, 0)),
            pl.BlockSpec((blk, H), lambda i: (i, 0)),
            pl.BlockSpec((blk, H), lambda i: (i, 0)),
            pl.BlockSpec((blk, H), lambda i: (i, 0)),
            pl.BlockSpec((blk, HC), lambda i: (i, 0)),
            pl.BlockSpec((1, HC), lambda i: (0, 0)),
        ],
        out_specs=pl.BlockSpec((blk, HC), lambda i: (i, 0)),
        out_shape=jax.ShapeDtypeStruct((N, HC), jnp.float32),
    )(accp, den_tot, a_src, a_dst, xw, bias)


def kernel(x, edge_index, edge_attr, W, att_src, att_dst, bias):
    del edge_attr
    src = edge_index[0].astype(jnp.int32)
    dst = edge_index[1].astype(jnp.int32)
    xw, a_src, a_dst = _tc_project(x, W, att_src, att_dst)
    pad = ((0, NPAD - N), (0, 0))
    a_srcT, a_dstT = _tc_transpose(jnp.pad(a_src, pad), jnp.pad(a_dst, pad),
                                   jnp.eye(H, dtype=jnp.float32))
    e_all, denp = _sc_phase1(a_srcT, a_dstT, src, dst)
    sd = jnp.stack([src.reshape(NW, NSB, SB), dst.reshape(NW, NSB, SB)],
                   axis=2)
    accp = _sc_phase2(xw, sd, e_all)
    den_tot = _tc_densum(denp.reshape(NW, NPAD * H // 128, 128))
    out = _tc_fin(accp, den_tot.reshape(NPAD, H), a_src, a_dst, xw,
                  bias.reshape(1, HC))
    return out
